# Initial kernel scaffold; baseline (speedup 1.0000x reference)
import jax
import jax.numpy as jnp
from jax.experimental import pallas as pl


def kernel(x, edge_index, W1, b1, W2, b2):
    raise NotImplementedError("write your pallas kernel here")


# speedup vs baseline: 17.6583x; 17.6583x over previous
import functools

import jax
import jax.numpy as jnp
from jax import lax
from jax.experimental import pallas as pl
from jax.experimental.pallas import tpu as pltpu
from jax.experimental.pallas import tpu_sc as plsc

N_NODES = 10000
N_EDGES = 320000
IN_DIM = 128
HID_DIM = 32

NC = 2
NS = 16
NW = NC * NS
N_PAD = 10240
STRIPE = N_PAD // NS
E_PER_W = N_EDGES // NW
CHUNK = 80
N_CHUNKS = E_PER_W // CHUNK

ROWB = 2048


def _sc_mesh():
    return plsc.VectorSubcoreMesh(core_axis_name="c", subcore_axis_name="s")


@functools.partial(
    pl.kernel,
    mesh=_sc_mesh(),
    out_type=jax.ShapeDtypeStruct((NC, N_PAD), jnp.float32),
    scratch_types=[
        pltpu.VMEM((CHUNK,), jnp.int32),
        pltpu.VMEM((CHUNK,), jnp.float32),
        pltpu.VMEM_SHARED((N_PAD,), jnp.float32),
    ],
    compiler_params=pltpu.CompilerParams(use_tc_tiling_on_sc=False),
)
def _deg_kernel(dst_hbm, ones_hbm, zeros_hbm, out_hbm, idx_d, ones_v, acc):
    c = lax.axis_index("c")
    s = lax.axis_index("s")
    row0 = pl.multiple_of(s * STRIPE, 8)
    pltpu.sync_copy(zeros_hbm, acc.at[pl.ds(row0, STRIPE)])
    pltpu.sync_copy(ones_hbm, ones_v)
    plsc.subcore_barrier()
    base = (c * NS + s) * E_PER_W

    def body(i, carry):
        off = pl.multiple_of(base + i * CHUNK, 8)
        pltpu.sync_copy(dst_hbm.at[pl.ds(off, CHUNK)], idx_d)
        pltpu.sync_copy(ones_v, acc.at[idx_d], add=True)
        return carry

    lax.fori_loop(0, N_CHUNKS, body, 0)
    plsc.subcore_barrier()
    pltpu.sync_copy(acc.at[pl.ds(row0, STRIPE)],
                    out_hbm.at[c, pl.ds(row0, STRIPE)])


def _make_agg(width):
    if width == 1:
        tab_t = jax.ShapeDtypeStruct((N_PAD,), jnp.float32)
        out_t = jax.ShapeDtypeStruct((NC, N_PAD), jnp.float32)
        rows_t = pltpu.VMEM((CHUNK,), jnp.float32)
        acc_t = pltpu.VMEM_SHARED((N_PAD,), jnp.float32)
    else:
        tab_t = jax.ShapeDtypeStruct((N_PAD, width), jnp.float32)
        out_t = jax.ShapeDtypeStruct((NC, N_PAD, width), jnp.float32)
        rows_t = pltpu.VMEM((CHUNK, width), jnp.float32)
        acc_t = pltpu.VMEM_SHARED((N_PAD, width), jnp.float32)
    del tab_t

    @functools.partial(
        pl.kernel,
        mesh=_sc_mesh(),
        out_type=out_t,
        scratch_types=[
            pltpu.VMEM((CHUNK,), jnp.int32),
            pltpu.VMEM((CHUNK,), jnp.int32),
            rows_t,
            acc_t,
            pltpu.SemaphoreType.DMA,
        ],
        compiler_params=pltpu.CompilerParams(use_tc_tiling_on_sc=False),
    )
    def agg_kernel(tab_hbm, src_hbm, dst_hbm, zeros_hbm, out_hbm,
                   idx_s, idx_d, rows, acc, sem):
        c = lax.axis_index("c")
        s = lax.axis_index("s")
        row0 = pl.multiple_of(s * STRIPE, 8)
        pltpu.sync_copy(zeros_hbm, acc.at[pl.ds(row0, STRIPE)])
        plsc.subcore_barrier()
        base = (c * NS + s) * E_PER_W

        def body(i, carry):
            off = pl.multiple_of(base + i * CHUNK, 8)
            pltpu.sync_copy(src_hbm.at[pl.ds(off, CHUNK)], idx_s)
            pltpu.sync_copy(dst_hbm.at[pl.ds(off, CHUNK)], idx_d)
            pltpu.async_copy(tab_hbm.at[idx_s], rows, sem).wait()
            pltpu.sync_copy(rows, acc.at[idx_d], add=True)
            return carry

        lax.fori_loop(0, N_CHUNKS, body, 0)
        plsc.subcore_barrier()
        pltpu.sync_copy(acc.at[pl.ds(row0, STRIPE)],
                        out_hbm.at[c, pl.ds(row0, STRIPE)])

    return agg_kernel


_agg32_kernel = _make_agg(HID_DIM)
_agg1_kernel = _make_agg(1)


def _t1_body(x_ref, w1_ref, degp_ref, y1_ref, dinv_ref):
    deg = degp_ref[0, :] + degp_ref[1, :] + 1.0
    dinv = lax.rsqrt(deg)
    h = jnp.dot(x_ref[...], w1_ref[...], preferred_element_type=jnp.float32)
    y1_ref[...] = h * dinv[:, None]
    dinv_ref[...] = dinv[:, None]


_t1 = pl.pallas_call(
    _t1_body,
    grid=(N_PAD // ROWB,),
    in_specs=[
        pl.BlockSpec((ROWB, IN_DIM), lambda i: (i, 0)),
        pl.BlockSpec((IN_DIM, HID_DIM), lambda i: (0, 0)),
        pl.BlockSpec((2, ROWB), lambda i: (0, i)),
    ],
    out_specs=[
        pl.BlockSpec((ROWB, HID_DIM), lambda i: (i, 0)),
        pl.BlockSpec((ROWB, 1), lambda i: (i, 0)),
    ],
    out_shape=[
        jax.ShapeDtypeStruct((N_PAD, HID_DIM), jnp.float32),
        jax.ShapeDtypeStruct((N_PAD, 1), jnp.float32),
    ],
)


def _t2_body(aggp_ref, y1_ref, dinv_ref, b1_ref, w2_ref, y2_ref):
    tot = aggp_ref[0] + aggp_ref[1] + y1_ref[...]
    dinv = dinv_ref[...]
    x2 = jnp.maximum(dinv * tot + b1_ref[...], 0.0)
    g = jnp.dot(x2, w2_ref[...], preferred_element_type=jnp.float32)
    y2_ref[...] = dinv * g


_t2 = pl.pallas_call(
    _t2_body,
    grid=(N_PAD // ROWB,),
    in_specs=[
        pl.BlockSpec((2, ROWB, HID_DIM), lambda i: (0, i, 0)),
        pl.BlockSpec((ROWB, HID_DIM), lambda i: (i, 0)),
        pl.BlockSpec((ROWB, 1), lambda i: (i, 0)),
        pl.BlockSpec((1, HID_DIM), lambda i: (0, 0)),
        pl.BlockSpec((HID_DIM, 1), lambda i: (0, 0)),
    ],
    out_specs=pl.BlockSpec((ROWB, 1), lambda i: (i, 0)),
    out_shape=jax.ShapeDtypeStruct((N_PAD, 1), jnp.float32),
)


def _t3_body(agg2p_ref, y2_ref, dinv_ref, b2_ref, out_ref):
    a = agg2p_ref[0, :][:, None] + agg2p_ref[1, :][:, None] + y2_ref[...]
    out_ref[...] = dinv_ref[...] * a + b2_ref[...]


_t3 = pl.pallas_call(
    _t3_body,
    grid=(N_PAD // ROWB,),
    in_specs=[
        pl.BlockSpec((2, ROWB), lambda i: (0, i)),
        pl.BlockSpec((ROWB, 1), lambda i: (i, 0)),
        pl.BlockSpec((ROWB, 1), lambda i: (i, 0)),
        pl.BlockSpec((1, 1), lambda i: (0, 0)),
    ],
    out_specs=pl.BlockSpec((ROWB, 1), lambda i: (i, 0)),
    out_shape=jax.ShapeDtypeStruct((N_PAD, 1), jnp.float32),
)


def kernel(x, edge_index, W1, b1, W2, b2):
    src = edge_index[0]
    dst = edge_index[1]
    x_p = jnp.pad(x, ((0, N_PAD - N_NODES), (0, 0)))
    zeros32 = jnp.zeros((STRIPE, HID_DIM), jnp.float32)
    zeros1 = jnp.zeros((STRIPE,), jnp.float32)
    ones_c = jnp.ones((CHUNK,), jnp.float32)

    degp = _deg_kernel(dst, ones_c, zeros1)
    y1, dinv = _t1(x_p, W1, degp)
    aggp = _agg32_kernel(y1, src, dst, zeros32)
    y2 = _t2(aggp, y1, dinv, b1.reshape(1, HID_DIM), W2)
    agg2p = _agg1_kernel(y2.reshape(-1), src, dst, zeros1)
    out = _t3(agg2p, y2, dinv, b2.reshape(1, 1))
    return out[:N_NODES, 0]


# speedup vs baseline: 45.8979x; 2.5992x over previous
import functools

import jax
import jax.numpy as jnp
from jax import lax
from jax.experimental import pallas as pl
from jax.experimental.pallas import tpu as pltpu
from jax.experimental.pallas import tpu_sc as plsc

N_NODES = 10000
N_EDGES = 320000
IN_DIM = 128
HID_DIM = 32

NC = 2
NS = 16
NW = NC * NS
N_PAD = 10240
STRIPE = N_PAD // NS
E_PER_W = N_EDGES // NW
EP_W = 10240

AGG_CHUNK = 128
AGG_NCH = EP_W // AGG_CHUNK
WAVE = 4
NWAVES = AGG_NCH // WAVE
NBUF = 2 * WAVE

SCL_CHUNK = 80
SCL_NCH = EP_W // SCL_CHUNK

ROWB = 2048


def _sc_mesh():
    return plsc.VectorSubcoreMesh(core_axis_name="c", subcore_axis_name="s")


_SC_PARAMS = pltpu.CompilerParams(use_tc_tiling_on_sc=False,
                                  needs_layout_passes=False)


@functools.partial(
    pl.kernel,
    mesh=_sc_mesh(),
    out_type=jax.ShapeDtypeStruct((NC, N_PAD), jnp.float32),
    scratch_types=[
        pltpu.VMEM((SCL_NCH, SCL_CHUNK), jnp.int32),
        pltpu.VMEM((SCL_CHUNK,), jnp.float32),
        pltpu.VMEM_SHARED((N_PAD,), jnp.float32),
        pltpu.SemaphoreType.DMA,
    ],
    compiler_params=_SC_PARAMS,
)
def _deg_kernel(dst_hbm, ones_hbm, zeros_hbm, out_hbm, idx_d, ones_v, acc, sem):
    c = lax.axis_index("c")
    s = lax.axis_index("s")
    wid = c * NS + s
    row0 = pl.multiple_of(s * STRIPE, 8)
    pltpu.sync_copy(zeros_hbm, acc.at[pl.ds(row0, STRIPE)])
    pltpu.sync_copy(ones_hbm, ones_v)
    pltpu.sync_copy(dst_hbm.at[wid], idx_d)
    plsc.subcore_barrier()

    def fire(i, carry):
        pltpu.async_copy(ones_v, acc.at[idx_d.at[i]], sem, add=True)
        return carry

    def drain(i, carry):
        pltpu.make_async_copy(ones_v, acc.at[idx_d.at[i]], sem).wait()
        return carry

    lax.fori_loop(0, SCL_NCH, fire, 0)
    lax.fori_loop(0, SCL_NCH, drain, 0)
    plsc.subcore_barrier()
    pltpu.sync_copy(acc.at[pl.ds(row0, STRIPE)],
                    out_hbm.at[c, pl.ds(row0, STRIPE)])


@functools.partial(
    pl.kernel,
    mesh=_sc_mesh(),
    out_type=jax.ShapeDtypeStruct((NC, N_PAD, HID_DIM), jnp.float32),
    scratch_types=[
        pltpu.VMEM((AGG_NCH, AGG_CHUNK), jnp.int32),
        pltpu.VMEM((AGG_NCH, AGG_CHUNK), jnp.int32),
        pltpu.VMEM((NBUF, AGG_CHUNK, HID_DIM), jnp.float32),
        pltpu.VMEM_SHARED((N_PAD, HID_DIM), jnp.float32),
        pltpu.SemaphoreType.DMA,
        pltpu.SemaphoreType.DMA,
    ],
    compiler_params=_SC_PARAMS,
)
def _agg32_kernel(tab_hbm, src_hbm, dst_hbm, zeros_hbm, out_hbm,
                  idx_s, idx_d, rows, acc, sem_g, sem_s):
    c = lax.axis_index("c")
    s = lax.axis_index("s")
    wid = c * NS + s
    row0 = pl.multiple_of(s * STRIPE, 8)
    pltpu.sync_copy(zeros_hbm, acc.at[pl.ds(row0, STRIPE)])
    pltpu.sync_copy(src_hbm.at[wid], idx_s)
    pltpu.sync_copy(dst_hbm.at[wid], idx_d)
    plsc.subcore_barrier()

    def fire_gather(chunk, slot):
        pltpu.async_copy(tab_hbm.at[idx_s.at[chunk]], rows.at[slot], sem_g)

    def drain_gather(chunk, slot):
        pltpu.make_async_copy(tab_hbm.at[idx_s.at[chunk]], rows.at[slot],
                              sem_g).wait()

    def fire_scatter(chunk, slot):
        pltpu.async_copy(rows.at[slot], acc.at[idx_d.at[chunk]], sem_s,
                         add=True)

    def drain_scatter(chunk, slot):
        pltpu.make_async_copy(rows.at[slot], acc.at[idx_d.at[chunk]],
                              sem_s).wait()

    for k in range(WAVE):
        fire_gather(k, k)

    def wave(w, carry):
        g = lax.rem(w, 2)
        slot0 = g * WAVE
        nslot0 = (1 - g) * WAVE
        base = w * WAVE

        @pl.when(w > 0)
        def _():
            for k in range(WAVE):
                drain_scatter(base - WAVE + k, nslot0 + k)

        @pl.when(w < NWAVES - 1)
        def _():
            for k in range(WAVE):
                fire_gather(base + WAVE + k, nslot0 + k)

        for k in range(WAVE):
            drain_gather(base + k, slot0 + k)
        for k in range(WAVE):
            fire_scatter(base + k, slot0 + k)
        return carry

    lax.fori_loop(0, NWAVES, wave, 0)
    last = NWAVES - 1
    for k in range(WAVE):
        drain_scatter(last * WAVE + k, (last % 2) * WAVE + k)
    plsc.subcore_barrier()
    pltpu.sync_copy(acc.at[pl.ds(row0, STRIPE)],
                    out_hbm.at[c, pl.ds(row0, STRIPE)])


@functools.partial(
    pl.kernel,
    mesh=_sc_mesh(),
    out_type=jax.ShapeDtypeStruct((NC, N_PAD), jnp.float32),
    scratch_types=[
        pltpu.VMEM((EP_W,), jnp.int32),
        pltpu.VMEM((SCL_NCH, SCL_CHUNK), jnp.int32),
        pltpu.VMEM((N_PAD,), jnp.float32),
        pltpu.VMEM((EP_W,), jnp.float32),
        pltpu.VMEM_SHARED((N_PAD,), jnp.float32),
        pltpu.SemaphoreType.DMA,
    ],
    compiler_params=_SC_PARAMS,
)
def _agg1_kernel(tab_hbm, srcf_hbm, dst_hbm, zeros_hbm, out_hbm,
                 idx_s, idx_d, tab_v, vals, acc, sem):
    c = lax.axis_index("c")
    s = lax.axis_index("s")
    wid = c * NS + s
    row0 = pl.multiple_of(s * STRIPE, 8)
    pltpu.sync_copy(zeros_hbm, acc.at[pl.ds(row0, STRIPE)])
    pltpu.sync_copy(tab_hbm, tab_v)
    pltpu.sync_copy(srcf_hbm.at[wid], idx_s)
    pltpu.sync_copy(dst_hbm.at[wid], idx_d)
    plsc.subcore_barrier()

    def gat(i, carry):
        base = i * SCL_CHUNK
        for j in range(SCL_CHUNK // 16):
            e = base + j * 16
            iv = idx_s[pl.ds(e, 16)]
            vals[pl.ds(e, 16)] = plsc.load_gather(tab_v, [iv])
        return carry

    lax.fori_loop(0, SCL_NCH, gat, 0)

    def fire(i, carry):
        pltpu.async_copy(vals.at[pl.ds(i * SCL_CHUNK, SCL_CHUNK)],
                         acc.at[idx_d.at[i]], sem, add=True)
        return carry

    def drain(i, carry):
        pltpu.make_async_copy(vals.at[pl.ds(i * SCL_CHUNK, SCL_CHUNK)],
                              acc.at[idx_d.at[i]], sem).wait()
        return carry

    lax.fori_loop(0, SCL_NCH, fire, 0)
    lax.fori_loop(0, SCL_NCH, drain, 0)
    plsc.subcore_barrier()
    pltpu.sync_copy(acc.at[pl.ds(row0, STRIPE)],
                    out_hbm.at[c, pl.ds(row0, STRIPE)])


def _t1_body(x_ref, w1_ref, degp_ref, y1_ref, dinv_ref):
    deg = degp_ref[0, :] + degp_ref[1, :] + 1.0
    dinv = lax.rsqrt(deg)
    h = jnp.dot(x_ref[...], w1_ref[...], preferred_element_type=jnp.float32)
    y1_ref[...] = h * dinv[:, None]
    dinv_ref[...] = dinv[:, None]


_t1 = pl.pallas_call(
    _t1_body,
    grid=(N_PAD // ROWB,),
    in_specs=[
        pl.BlockSpec((ROWB, IN_DIM), lambda i: (i, 0)),
        pl.BlockSpec((IN_DIM, HID_DIM), lambda i: (0, 0)),
        pl.BlockSpec((2, ROWB), lambda i: (0, i)),
    ],
    out_specs=[
        pl.BlockSpec((ROWB, HID_DIM), lambda i: (i, 0)),
        pl.BlockSpec((ROWB, 1), lambda i: (i, 0)),
    ],
    out_shape=[
        jax.ShapeDtypeStruct((N_PAD, HID_DIM), jnp.float32),
        jax.ShapeDtypeStruct((N_PAD, 1), jnp.float32),
    ],
)


def _t2_body(aggp_ref, y1_ref, dinv_ref, b1_ref, w2_ref, y2_ref):
    tot = aggp_ref[0] + aggp_ref[1] + y1_ref[...]
    dinv = dinv_ref[...]
    x2 = jnp.maximum(dinv * tot + b1_ref[...], 0.0)
    g = jnp.dot(x2, w2_ref[...], preferred_element_type=jnp.float32)
    y2_ref[...] = dinv * g


_t2 = pl.pallas_call(
    _t2_body,
    grid=(N_PAD // ROWB,),
    in_specs=[
        pl.BlockSpec((2, ROWB, HID_DIM), lambda i: (0, i, 0)),
        pl.BlockSpec((ROWB, HID_DIM), lambda i: (i, 0)),
        pl.BlockSpec((ROWB, 1), lambda i: (i, 0)),
        pl.BlockSpec((1, HID_DIM), lambda i: (0, 0)),
        pl.BlockSpec((HID_DIM, 1), lambda i: (0, 0)),
    ],
    out_specs=pl.BlockSpec((ROWB, 1), lambda i: (i, 0)),
    out_shape=jax.ShapeDtypeStruct((N_PAD, 1), jnp.float32),
)


def _t3_body(agg2p_ref, y2_ref, dinv_ref, b2_ref, out_ref):
    a = agg2p_ref[0, :][:, None] + agg2p_ref[1, :][:, None] + y2_ref[...]
    out_ref[...] = dinv_ref[...] * a + b2_ref[...]


_t3 = pl.pallas_call(
    _t3_body,
    grid=(N_PAD // ROWB,),
    in_specs=[
        pl.BlockSpec((2, ROWB), lambda i: (0, i)),
        pl.BlockSpec((ROWB, 1), lambda i: (i, 0)),
        pl.BlockSpec((ROWB, 1), lambda i: (i, 0)),
        pl.BlockSpec((1, 1), lambda i: (0, 0)),
    ],
    out_specs=pl.BlockSpec((ROWB, 1), lambda i: (i, 0)),
    out_shape=jax.ShapeDtypeStruct((N_PAD, 1), jnp.float32),
)


def kernel(x, edge_index, W1, b1, W2, b2):
    pad = jnp.full((NW, EP_W - E_PER_W), N_PAD - 1, jnp.int32)
    src_p = jnp.concatenate([edge_index[0].reshape(NW, E_PER_W), pad], axis=1)
    dst_p = jnp.concatenate([edge_index[1].reshape(NW, E_PER_W), pad], axis=1)
    src_a = src_p.reshape(NW, AGG_NCH, AGG_CHUNK)
    dst_a = dst_p.reshape(NW, AGG_NCH, AGG_CHUNK)
    dst_b = dst_p.reshape(NW, SCL_NCH, SCL_CHUNK)

    x_p = jnp.pad(x, ((0, N_PAD - N_NODES), (0, 0)))
    zeros32 = jnp.zeros((STRIPE, HID_DIM), jnp.float32)
    zeros1 = jnp.zeros((STRIPE,), jnp.float32)
    ones_c = jnp.ones((SCL_CHUNK,), jnp.float32)

    degp = _deg_kernel(dst_b, ones_c, zeros1)
    y1, dinv = _t1(x_p, W1, degp)
    aggp = _agg32_kernel(y1, src_a, dst_a, zeros32)
    y2 = _t2(aggp, y1, dinv, b1.reshape(1, HID_DIM), W2)
    agg2p = _agg1_kernel(y2.reshape(-1), src_p, dst_b, zeros1)
    out = _t3(agg2p, y2, dinv, b2.reshape(1, 1))
    return out[:N_NODES, 0]


# speedup vs baseline: 71.7573x; 1.5634x over previous
import functools

import jax
import jax.numpy as jnp
from jax import lax
from jax.experimental import pallas as pl
from jax.experimental.pallas import tpu as pltpu
from jax.experimental.pallas import tpu_sc as plsc

N_NODES = 10000
N_EDGES = 320000
IN_DIM = 128
HID_DIM = 32

NC = 2
NS = 16
NW = NC * NS
N_PAD = 10240
STRIPE = N_PAD // NS
E_PER_W = N_EDGES // NW

CHUNK = 80
NCH = E_PER_W // CHUNK
WAVE = 5
NWAVES = NCH // WAVE
NBUF = 2 * WAVE


def _sc_mesh():
    return plsc.VectorSubcoreMesh(core_axis_name="c", subcore_axis_name="s")


_SC_PARAMS = pltpu.CompilerParams(use_tc_tiling_on_sc=False,
                                  needs_layout_passes=False)


@functools.partial(
    pl.kernel,
    mesh=_sc_mesh(),
    out_type=jax.ShapeDtypeStruct((NC, N_PAD), jnp.float32),
    scratch_types=[
        pltpu.VMEM((NCH, CHUNK), jnp.int32),
        pltpu.VMEM((CHUNK,), jnp.float32),
        pltpu.VMEM_SHARED((N_PAD,), jnp.float32),
        pltpu.SemaphoreType.DMA,
    ],
    compiler_params=_SC_PARAMS,
)
def _deg_kernel(dst_hbm, ones_hbm, zeros_hbm, out_hbm, idx_d, ones_v, acc, sem):
    c = lax.axis_index("c")
    s = lax.axis_index("s")
    wid = c * NS + s
    row0 = pl.multiple_of(s * STRIPE, 8)
    pltpu.sync_copy(zeros_hbm, acc.at[pl.ds(row0, STRIPE)])
    pltpu.sync_copy(ones_hbm, ones_v)
    pltpu.sync_copy(dst_hbm.at[wid], idx_d)
    plsc.subcore_barrier()

    def fire(i, carry):
        pltpu.async_copy(ones_v, acc.at[idx_d.at[i]], sem, add=True)
        return carry

    def drain(i, carry):
        pltpu.make_async_copy(ones_v, acc.at[idx_d.at[i]], sem).wait()
        return carry

    lax.fori_loop(0, NCH, fire, 0)
    lax.fori_loop(0, NCH, drain, 0)
    plsc.subcore_barrier()
    pltpu.sync_copy(acc.at[pl.ds(row0, STRIPE)],
                    out_hbm.at[c, pl.ds(row0, STRIPE)])


@functools.partial(
    pl.kernel,
    mesh=_sc_mesh(),
    out_type=jax.ShapeDtypeStruct((NC, N_PAD, HID_DIM), jnp.float32),
    scratch_types=[
        pltpu.VMEM((NCH, CHUNK), jnp.int32),
        pltpu.VMEM((NCH, CHUNK), jnp.int32),
        pltpu.VMEM((NBUF, CHUNK, HID_DIM), jnp.float32),
        pltpu.VMEM_SHARED((N_PAD, HID_DIM), jnp.float32),
        pltpu.SemaphoreType.DMA,
        pltpu.SemaphoreType.DMA,
    ],
    compiler_params=_SC_PARAMS,
)
def _agg32_kernel(tab_hbm, src_hbm, dst_hbm, zeros_hbm, out_hbm,
                  idx_s, idx_d, rows, acc, sem_g, sem_s):
    c = lax.axis_index("c")
    s = lax.axis_index("s")
    wid = c * NS + s
    row0 = pl.multiple_of(s * STRIPE, 8)
    pltpu.sync_copy(zeros_hbm, acc.at[pl.ds(row0, STRIPE)])
    pltpu.sync_copy(src_hbm.at[wid], idx_s)
    pltpu.sync_copy(dst_hbm.at[wid], idx_d)
    plsc.subcore_barrier()

    def fire_gather(chunk, slot):
        pltpu.async_copy(tab_hbm.at[idx_s.at[chunk]], rows.at[slot], sem_g)

    def drain_gather(chunk, slot):
        pltpu.make_async_copy(tab_hbm.at[idx_s.at[chunk]], rows.at[slot],
                              sem_g).wait()

    def fire_scatter(chunk, slot):
        pltpu.async_copy(rows.at[slot], acc.at[idx_d.at[chunk]], sem_s,
                         add=True)

    def drain_scatter(chunk, slot):
        pltpu.make_async_copy(rows.at[slot], acc.at[idx_d.at[chunk]],
                              sem_s).wait()

    for k in range(WAVE):
        fire_gather(k, k)

    def wave(w, carry):
        g = lax.rem(w, 2)
        slot0 = g * WAVE
        nslot0 = (1 - g) * WAVE
        base = w * WAVE

        @pl.when(w > 0)
        def _():
            for k in range(WAVE):
                drain_scatter(base - WAVE + k, nslot0 + k)

        @pl.when(w < NWAVES - 1)
        def _():
            for k in range(WAVE):
                fire_gather(base + WAVE + k, nslot0 + k)

        for k in range(WAVE):
            drain_gather(base + k, slot0 + k)
        for k in range(WAVE):
            fire_scatter(base + k, slot0 + k)
        return carry

    lax.fori_loop(0, NWAVES, wave, 0)
    last = NWAVES - 1
    for k in range(WAVE):
        drain_scatter(last * WAVE + k, (last % 2) * WAVE + k)
    plsc.subcore_barrier()
    pltpu.sync_copy(acc.at[pl.ds(row0, STRIPE)],
                    out_hbm.at[c, pl.ds(row0, STRIPE)])


@functools.partial(
    pl.kernel,
    mesh=_sc_mesh(),
    out_type=jax.ShapeDtypeStruct((NC, N_PAD), jnp.float32),
    scratch_types=[
        pltpu.VMEM((E_PER_W,), jnp.int32),
        pltpu.VMEM((NCH, CHUNK), jnp.int32),
        pltpu.VMEM((N_NODES,), jnp.float32),
        pltpu.VMEM((E_PER_W,), jnp.float32),
        pltpu.VMEM_SHARED((N_PAD,), jnp.float32),
        pltpu.SemaphoreType.DMA,
    ],
    compiler_params=_SC_PARAMS,
)
def _agg1_kernel(tab_hbm, srcf_hbm, dst_hbm, zeros_hbm, out_hbm,
                 idx_s, idx_d, tab_v, vals, acc, sem):
    c = lax.axis_index("c")
    s = lax.axis_index("s")
    wid = c * NS + s
    row0 = pl.multiple_of(s * STRIPE, 8)
    pltpu.sync_copy(zeros_hbm, acc.at[pl.ds(row0, STRIPE)])
    pltpu.sync_copy(tab_hbm, tab_v)
    pltpu.sync_copy(srcf_hbm.at[wid], idx_s)
    pltpu.sync_copy(dst_hbm.at[wid], idx_d)
    plsc.subcore_barrier()

    def gat(i, carry):
        base = i * CHUNK
        for j in range(CHUNK // 16):
            e = base + j * 16
            iv = idx_s[pl.ds(e, 16)]
            vals[pl.ds(e, 16)] = plsc.load_gather(tab_v, [iv])
        return carry

    lax.fori_loop(0, NCH, gat, 0)

    def fire(i, carry):
        pltpu.async_copy(vals.at[pl.ds(i * CHUNK, CHUNK)],
                         acc.at[idx_d.at[i]], sem, add=True)
        return carry

    def drain(i, carry):
        pltpu.make_async_copy(vals.at[pl.ds(i * CHUNK, CHUNK)],
                              acc.at[idx_d.at[i]], sem).wait()
        return carry

    lax.fori_loop(0, NCH, fire, 0)
    lax.fori_loop(0, NCH, drain, 0)
    plsc.subcore_barrier()
    pltpu.sync_copy(acc.at[pl.ds(row0, STRIPE)],
                    out_hbm.at[c, pl.ds(row0, STRIPE)])


def _t1_body(x_ref, w1_ref, degp_ref, y1_ref, dinv_ref):
    deg = degp_ref[0, :N_NODES] + degp_ref[1, :N_NODES] + 1.0
    dinv = lax.rsqrt(deg)
    h = jnp.dot(x_ref[...], w1_ref[...], preferred_element_type=jnp.float32)
    y1_ref[...] = h * dinv[:, None]
    dinv_ref[...] = dinv[:, None]


_t1 = pl.pallas_call(
    _t1_body,
    out_shape=[
        jax.ShapeDtypeStruct((N_NODES, HID_DIM), jnp.float32),
        jax.ShapeDtypeStruct((N_NODES, 1), jnp.float32),
    ],
)


def _t2_body(aggp_ref, y1_ref, dinv_ref, b1_ref, w2_ref, y2_ref):
    tot = aggp_ref[0, :N_NODES, :] + aggp_ref[1, :N_NODES, :] + y1_ref[...]
    dinv = dinv_ref[...]
    x2 = jnp.maximum(dinv * tot + b1_ref[...], 0.0)
    g = jnp.dot(x2, w2_ref[...], preferred_element_type=jnp.float32)
    y2_ref[...] = dinv * g


_t2 = pl.pallas_call(
    _t2_body,
    out_shape=jax.ShapeDtypeStruct((N_NODES, 1), jnp.float32),
)


def _t3_body(agg2p_ref, y2_ref, dinv_ref, b2_ref, out_ref):
    a = (agg2p_ref[0, :N_NODES][:, None] + agg2p_ref[1, :N_NODES][:, None]
         + y2_ref[...])
    out_ref[...] = dinv_ref[...] * a + b2_ref[...]


_t3 = pl.pallas_call(
    _t3_body,
    out_shape=jax.ShapeDtypeStruct((N_NODES, 1), jnp.float32),
)


def kernel(x, edge_index, W1, b1, W2, b2):
    src_f = edge_index[0].reshape(NW, E_PER_W)
    dst_f = edge_index[1].reshape(NW, E_PER_W)
    src_c = src_f.reshape(NW, NCH, CHUNK)
    dst_c = dst_f.reshape(NW, NCH, CHUNK)

    zeros32 = jnp.zeros((STRIPE, HID_DIM), jnp.float32)
    zeros1 = jnp.zeros((STRIPE,), jnp.float32)
    ones_c = jnp.ones((CHUNK,), jnp.float32)

    degp = _deg_kernel(dst_c, ones_c, zeros1)
    y1, dinv = _t1(x, W1, degp)
    aggp = _agg32_kernel(y1, src_c, dst_c, zeros32)
    y2 = _t2(aggp, y1, dinv, b1.reshape(1, HID_DIM), W2)
    agg2p = _agg1_kernel(y2.reshape(-1), src_f, dst_c, zeros1)
    out = _t3(agg2p, y2, dinv, b2.reshape(1, 1))
    return out[:, 0]


# speedup vs baseline: 83.8282x; 1.1682x over previous
import functools

import jax
import jax.numpy as jnp
from jax import lax
from jax.experimental import pallas as pl
from jax.experimental.pallas import tpu as pltpu
from jax.experimental.pallas import tpu_sc as plsc

N_NODES = 10000
N_EDGES = 320000
IN_DIM = 128
HID_DIM = 32

NC = 2
NS = 16
NW = NC * NS
N_PAD = 10240
STRIPE = N_PAD // NS
E_PER_W = N_EDGES // NW

CHUNK = 80
NCH = E_PER_W // CHUNK
WAVE = 5
NWAVES = NCH // WAVE
NBUF = 2 * WAVE


def _sc_mesh():
    return plsc.VectorSubcoreMesh(core_axis_name="c", subcore_axis_name="s")


_SC_PARAMS = pltpu.CompilerParams(use_tc_tiling_on_sc=False,
                                  needs_layout_passes=False)


@functools.partial(
    pl.kernel,
    mesh=_sc_mesh(),
    out_type=jax.ShapeDtypeStruct((NC, N_PAD), jnp.float32),
    scratch_types=[
        pltpu.VMEM((NCH, CHUNK), jnp.int32),
        pltpu.VMEM((CHUNK,), jnp.float32),
        pltpu.VMEM_SHARED((N_PAD,), jnp.float32),
        pltpu.SemaphoreType.DMA,
    ],
    compiler_params=_SC_PARAMS,
)
def _deg_kernel(ei_hbm, ones_hbm, zeros_hbm, out_hbm, idx_d, ones_v, acc, sem):
    c = lax.axis_index("c")
    s = lax.axis_index("s")
    wid = c * NS + s
    row0 = pl.multiple_of(s * STRIPE, 8)
    pltpu.sync_copy(zeros_hbm, acc.at[pl.ds(row0, STRIPE)])
    pltpu.sync_copy(ones_hbm, ones_v)
    pltpu.sync_copy(ei_hbm.at[1, wid], idx_d)
    plsc.subcore_barrier()

    def fire(i, carry):
        pltpu.async_copy(ones_v, acc.at[idx_d.at[i]], sem, add=True)
        return carry

    def drain(i, carry):
        pltpu.make_async_copy(ones_v, acc.at[idx_d.at[i]], sem).wait()
        return carry

    lax.fori_loop(0, NCH, fire, 0)
    lax.fori_loop(0, NCH, drain, 0)
    plsc.subcore_barrier()
    pltpu.sync_copy(acc.at[pl.ds(row0, STRIPE)],
                    out_hbm.at[c, pl.ds(row0, STRIPE)])


@functools.partial(
    pl.kernel,
    mesh=_sc_mesh(),
    out_type=jax.ShapeDtypeStruct((NC, N_PAD, HID_DIM), jnp.float32),
    scratch_types=[
        pltpu.VMEM((NCH, CHUNK), jnp.int32),
        pltpu.VMEM((NCH, CHUNK), jnp.int32),
        pltpu.VMEM((NBUF, CHUNK, HID_DIM), jnp.float32),
        pltpu.VMEM_SHARED((N_PAD, HID_DIM), jnp.float32),
        pltpu.SemaphoreType.DMA,
        pltpu.SemaphoreType.DMA,
    ],
    compiler_params=_SC_PARAMS,
)
def _agg32_kernel(tab_hbm, ei_hbm, zeros_hbm, out_hbm,
                  idx_s, idx_d, rows, acc, sem_g, sem_s):
    c = lax.axis_index("c")
    s = lax.axis_index("s")
    wid = c * NS + s
    row0 = pl.multiple_of(s * STRIPE, 8)
    pltpu.sync_copy(zeros_hbm, acc.at[pl.ds(row0, STRIPE)])
    pltpu.sync_copy(ei_hbm.at[0, wid], idx_s)
    pltpu.sync_copy(ei_hbm.at[1, wid], idx_d)
    plsc.subcore_barrier()

    def fire_gather(chunk, slot):
        pltpu.async_copy(tab_hbm.at[idx_s.at[chunk]], rows.at[slot], sem_g)

    def drain_gather(chunk, slot):
        pltpu.make_async_copy(tab_hbm.at[idx_s.at[chunk]], rows.at[slot],
                              sem_g).wait()

    def fire_scatter(chunk, slot):
        pltpu.async_copy(rows.at[slot], acc.at[idx_d.at[chunk]], sem_s,
                         add=True)

    def drain_scatter(chunk, slot):
        pltpu.make_async_copy(rows.at[slot], acc.at[idx_d.at[chunk]],
                              sem_s).wait()

    for k in range(WAVE):
        fire_gather(k, k)

    def wave(w, carry):
        g = lax.rem(w, 2)
        slot0 = g * WAVE
        nslot0 = (1 - g) * WAVE
        base = w * WAVE

        @pl.when(w > 0)
        def _():
            for k in range(WAVE):
                drain_scatter(base - WAVE + k, nslot0 + k)

        @pl.when(w < NWAVES - 1)
        def _():
            for k in range(WAVE):
                fire_gather(base + WAVE + k, nslot0 + k)

        for k in range(WAVE):
            drain_gather(base + k, slot0 + k)
        for k in range(WAVE):
            fire_scatter(base + k, slot0 + k)
        return carry

    lax.fori_loop(0, NWAVES, wave, 0)
    last = NWAVES - 1
    for k in range(WAVE):
        drain_scatter(last * WAVE + k, (last % 2) * WAVE + k)
    plsc.subcore_barrier()
    pltpu.sync_copy(acc.at[pl.ds(row0, STRIPE)],
                    out_hbm.at[c, pl.ds(row0, STRIPE)])


@functools.partial(
    pl.kernel,
    mesh=_sc_mesh(),
    out_type=jax.ShapeDtypeStruct((NC, N_PAD), jnp.float32),
    scratch_types=[
        pltpu.VMEM((NCH, CHUNK), jnp.int32),
        pltpu.VMEM((NCH, CHUNK), jnp.int32),
        pltpu.VMEM((N_NODES,), jnp.float32),
        pltpu.VMEM((E_PER_W,), jnp.float32),
        pltpu.VMEM_SHARED((N_PAD,), jnp.float32),
        pltpu.SemaphoreType.DMA,
    ],
    compiler_params=_SC_PARAMS,
)
def _agg1_kernel(tab_hbm, ei_hbm, zeros_hbm, out_hbm,
                 idx_s, idx_d, tab_v, vals, acc, sem):
    c = lax.axis_index("c")
    s = lax.axis_index("s")
    wid = c * NS + s
    row0 = pl.multiple_of(s * STRIPE, 8)
    pltpu.sync_copy(zeros_hbm, acc.at[pl.ds(row0, STRIPE)])
    pltpu.sync_copy(tab_hbm, tab_v)
    pltpu.sync_copy(ei_hbm.at[0, wid], idx_s)
    pltpu.sync_copy(ei_hbm.at[1, wid], idx_d)
    plsc.subcore_barrier()

    def gat(i, carry):
        for j in range(CHUNK // 16):
            iv = idx_s[i, pl.ds(j * 16, 16)]
            vals[pl.ds(i * CHUNK + j * 16, 16)] = plsc.load_gather(tab_v, [iv])
        return carry

    lax.fori_loop(0, NCH, gat, 0)

    def fire(i, carry):
        pltpu.async_copy(vals.at[pl.ds(i * CHUNK, CHUNK)],
                         acc.at[idx_d.at[i]], sem, add=True)
        return carry

    def drain(i, carry):
        pltpu.make_async_copy(vals.at[pl.ds(i * CHUNK, CHUNK)],
                              acc.at[idx_d.at[i]], sem).wait()
        return carry

    lax.fori_loop(0, NCH, fire, 0)
    lax.fori_loop(0, NCH, drain, 0)
    plsc.subcore_barrier()
    pltpu.sync_copy(acc.at[pl.ds(row0, STRIPE)],
                    out_hbm.at[c, pl.ds(row0, STRIPE)])


def _t1_body(x_ref, w1_ref, degp_ref, y1_ref, dinv_ref):
    deg = degp_ref[0, :N_NODES] + degp_ref[1, :N_NODES] + 1.0
    dinv = lax.rsqrt(deg)
    h = jnp.dot(x_ref[...], w1_ref[...], preferred_element_type=jnp.float32)
    y1_ref[...] = h * dinv[:, None]
    dinv_ref[...] = dinv[:, None]


_t1 = pl.pallas_call(
    _t1_body,
    out_shape=[
        jax.ShapeDtypeStruct((N_NODES, HID_DIM), jnp.float32),
        jax.ShapeDtypeStruct((N_NODES, 1), jnp.float32),
    ],
)


def _t2_body(aggp_ref, y1_ref, dinv_ref, b1_ref, w2_ref, y2_ref):
    tot = aggp_ref[0, :N_NODES, :] + aggp_ref[1, :N_NODES, :] + y1_ref[...]
    dinv = dinv_ref[...]
    x2 = jnp.maximum(dinv * tot + b1_ref[...], 0.0)
    g = jnp.dot(x2, w2_ref[...], preferred_element_type=jnp.float32)
    y2_ref[...] = (dinv * g)[:, 0]


_t2 = pl.pallas_call(
    _t2_body,
    out_shape=jax.ShapeDtypeStruct((N_NODES,), jnp.float32),
)


def _t3_body(agg2p_ref, y2_ref, dinv_ref, b2_ref, out_ref):
    a = agg2p_ref[0, :N_NODES] + agg2p_ref[1, :N_NODES] + y2_ref[...]
    out_ref[...] = dinv_ref[:, 0] * a + b2_ref[0, 0]


_t3 = pl.pallas_call(
    _t3_body,
    out_shape=jax.ShapeDtypeStruct((N_NODES,), jnp.float32),
)


def kernel(x, edge_index, W1, b1, W2, b2):
    ei4 = edge_index.reshape(2, NW, NCH, CHUNK)

    zeros32 = jnp.zeros((STRIPE, HID_DIM), jnp.float32)
    zeros1 = jnp.zeros((STRIPE,), jnp.float32)
    ones_c = jnp.ones((CHUNK,), jnp.float32)

    degp = _deg_kernel(ei4, ones_c, zeros1)
    y1, dinv = _t1(x, W1, degp)
    aggp = _agg32_kernel(y1, ei4, zeros32)
    y2 = _t2(aggp, y1, dinv, b1.reshape(1, HID_DIM), W2)
    agg2p = _agg1_kernel(y2, ei4, zeros1)
    out = _t3(agg2p, y2, dinv, b2.reshape(1, 1))
    return out


# speedup vs baseline: 87.1008x; 1.0390x over previous
import functools

import jax
import jax.numpy as jnp
from jax import lax
from jax.experimental import pallas as pl
from jax.experimental.pallas import tpu as pltpu
from jax.experimental.pallas import tpu_sc as plsc

N_NODES = 10000
N_EDGES = 320000
IN_DIM = 128
HID_DIM = 32

NC = 2
NS = 16
NW = NC * NS
N_PAD = 10240
STRIPE = N_PAD // NS
E_PER_W = N_EDGES // NW

CHUNK = 80
NCH = E_PER_W // CHUNK
WAVE = 5
NWAVES = NCH // WAVE
NGRP = 3
NBUF = NGRP * WAVE


def _sc_mesh():
    return plsc.VectorSubcoreMesh(core_axis_name="c", subcore_axis_name="s")


_SC_PARAMS = pltpu.CompilerParams(use_tc_tiling_on_sc=False,
                                  needs_layout_passes=False)


@functools.partial(
    pl.kernel,
    mesh=_sc_mesh(),
    out_type=jax.ShapeDtypeStruct((NC, N_PAD), jnp.float32),
    scratch_types=[
        pltpu.VMEM((NCH, CHUNK), jnp.int32),
        pltpu.VMEM((CHUNK,), jnp.float32),
        pltpu.VMEM_SHARED((N_PAD,), jnp.float32),
        pltpu.SemaphoreType.DMA,
    ],
    compiler_params=_SC_PARAMS,
)
def _deg_kernel(ei_hbm, ones_hbm, zeros_hbm, out_hbm, idx_d, ones_v, acc, sem):
    c = lax.axis_index("c")
    s = lax.axis_index("s")
    wid = c * NS + s
    row0 = pl.multiple_of(s * STRIPE, 8)
    pltpu.sync_copy(zeros_hbm, acc.at[pl.ds(row0, STRIPE)])
    pltpu.sync_copy(ones_hbm, ones_v)
    pltpu.sync_copy(ei_hbm.at[1, wid], idx_d)
    plsc.subcore_barrier()

    def fire(i, carry):
        pltpu.async_copy(ones_v, acc.at[idx_d.at[i]], sem, add=True)
        return carry

    def drain(i, carry):
        pltpu.make_async_copy(ones_v, acc.at[idx_d.at[i]], sem).wait()
        return carry

    lax.fori_loop(0, NCH, fire, 0)
    lax.fori_loop(0, NCH, drain, 0)
    plsc.subcore_barrier()
    pltpu.sync_copy(acc.at[pl.ds(row0, STRIPE)],
                    out_hbm.at[c, pl.ds(row0, STRIPE)])


@functools.partial(
    pl.kernel,
    mesh=_sc_mesh(),
    out_type=jax.ShapeDtypeStruct((NC, N_PAD, HID_DIM), jnp.float32),
    scratch_types=[
        pltpu.VMEM((NCH, CHUNK), jnp.int32),
        pltpu.VMEM((NCH, CHUNK), jnp.int32),
        pltpu.VMEM((NBUF, CHUNK, HID_DIM), jnp.float32),
        pltpu.VMEM_SHARED((N_PAD, HID_DIM), jnp.float32),
        pltpu.SemaphoreType.DMA,
        pltpu.SemaphoreType.DMA,
    ],
    compiler_params=_SC_PARAMS,
)
def _agg32_kernel(tab_hbm, ei_hbm, zeros_hbm, out_hbm,
                  idx_s, idx_d, rows, acc, sem_g, sem_s):
    c = lax.axis_index("c")
    s = lax.axis_index("s")
    wid = c * NS + s
    row0 = pl.multiple_of(s * STRIPE, 8)
    pltpu.sync_copy(zeros_hbm, acc.at[pl.ds(row0, STRIPE)])
    pltpu.sync_copy(ei_hbm.at[0, wid], idx_s)
    pltpu.sync_copy(ei_hbm.at[1, wid], idx_d)
    plsc.subcore_barrier()

    def fire_gather(chunk, slot):
        pltpu.async_copy(tab_hbm.at[idx_s.at[chunk]], rows.at[slot], sem_g)

    def drain_gather(chunk, slot):
        pltpu.make_async_copy(tab_hbm.at[idx_s.at[chunk]], rows.at[slot],
                              sem_g).wait()

    def fire_scatter(chunk, slot):
        pltpu.async_copy(rows.at[slot], acc.at[idx_d.at[chunk]], sem_s,
                         add=True)

    def drain_scatter(chunk, slot):
        pltpu.make_async_copy(rows.at[slot], acc.at[idx_d.at[chunk]],
                              sem_s).wait()

    for k in range(WAVE):
        fire_gather(k, k)

    def wave(w, carry):
        g = lax.rem(w, NGRP)
        ng = lax.rem(w + 1, NGRP)
        slot0 = g * WAVE
        nslot0 = ng * WAVE
        base = w * WAVE

        @pl.when(w > 1)
        def _():
            for k in range(WAVE):
                drain_scatter(base - 2 * WAVE + k, nslot0 + k)

        @pl.when(w < NWAVES - 1)
        def _():
            for k in range(WAVE):
                fire_gather(base + WAVE + k, nslot0 + k)

        for k in range(WAVE):
            drain_gather(base + k, slot0 + k)
        for k in range(WAVE):
            fire_scatter(base + k, slot0 + k)
        return carry

    lax.fori_loop(0, NWAVES, wave, 0)
    for w in (NWAVES - 2, NWAVES - 1):
        for k in range(WAVE):
            drain_scatter(w * WAVE + k, (w % NGRP) * WAVE + k)
    plsc.subcore_barrier()
    pltpu.sync_copy(acc.at[pl.ds(row0, STRIPE)],
                    out_hbm.at[c, pl.ds(row0, STRIPE)])


@functools.partial(
    pl.kernel,
    mesh=_sc_mesh(),
    out_type=jax.ShapeDtypeStruct((NC, N_PAD), jnp.float32),
    scratch_types=[
        pltpu.VMEM((NCH, CHUNK), jnp.int32),
        pltpu.VMEM((NCH, CHUNK), jnp.int32),
        pltpu.VMEM((N_PAD,), jnp.float32),
        pltpu.VMEM((E_PER_W,), jnp.float32),
        pltpu.VMEM_SHARED((N_PAD,), jnp.float32),
        pltpu.SemaphoreType.DMA,
    ],
    compiler_params=_SC_PARAMS,
)
def _agg1_kernel(tab_hbm, ei_hbm, zeros_hbm, out_hbm,
                 idx_s, idx_d, tab_v, vals, acc, sem):
    c = lax.axis_index("c")
    s = lax.axis_index("s")
    wid = c * NS + s
    row0 = pl.multiple_of(s * STRIPE, 8)
    pltpu.sync_copy(zeros_hbm, acc.at[pl.ds(row0, STRIPE)])
    pltpu.sync_copy(tab_hbm, tab_v)
    pltpu.sync_copy(ei_hbm.at[0, wid], idx_s)
    pltpu.sync_copy(ei_hbm.at[1, wid], idx_d)
    plsc.subcore_barrier()

    def gat(i, carry):
        for j in range(CHUNK // 16):
            iv = idx_s[i, pl.ds(j * 16, 16)]
            vals[pl.ds(i * CHUNK + j * 16, 16)] = plsc.load_gather(tab_v, [iv])
        return carry

    lax.fori_loop(0, NCH, gat, 0)

    def fire(i, carry):
        pltpu.async_copy(vals.at[pl.ds(i * CHUNK, CHUNK)],
                         acc.at[idx_d.at[i]], sem, add=True)
        return carry

    def drain(i, carry):
        pltpu.make_async_copy(vals.at[pl.ds(i * CHUNK, CHUNK)],
                              acc.at[idx_d.at[i]], sem).wait()
        return carry

    lax.fori_loop(0, NCH, fire, 0)
    lax.fori_loop(0, NCH, drain, 0)
    plsc.subcore_barrier()
    pltpu.sync_copy(acc.at[pl.ds(row0, STRIPE)],
                    out_hbm.at[c, pl.ds(row0, STRIPE)])


def _t1_body(x_ref, w1_ref, degp_ref, y1_ref, dinv_ref):
    deg = degp_ref[0, :N_NODES] + degp_ref[1, :N_NODES] + 1.0
    dinv = lax.rsqrt(deg)
    h = jnp.dot(x_ref[...], w1_ref[...], preferred_element_type=jnp.float32)
    y1_ref[...] = h * dinv[:, None]
    dinv_ref[...] = dinv[:, None]


_t1 = pl.pallas_call(
    _t1_body,
    out_shape=[
        jax.ShapeDtypeStruct((N_NODES, HID_DIM), jnp.float32),
        jax.ShapeDtypeStruct((N_NODES, 1), jnp.float32),
    ],
)


def _t2_body(aggp_ref, y1_ref, dinv_ref, b1_ref, w2_ref, y2_ref):
    tot = aggp_ref[0] + aggp_ref[1] + y1_ref[...]
    dinv = dinv_ref[...]
    x2 = jnp.maximum(dinv * tot + b1_ref[...], 0.0)
    g = jnp.dot(x2, w2_ref[...], preferred_element_type=jnp.float32)
    y2_ref[...] = (dinv * g)[:, 0]


_T2B = 2048
_t2 = pl.pallas_call(
    _t2_body,
    grid=(N_PAD // _T2B,),
    in_specs=[
        pl.BlockSpec((2, _T2B, HID_DIM), lambda i: (0, i, 0)),
        pl.BlockSpec((_T2B, HID_DIM), lambda i: (i, 0)),
        pl.BlockSpec((_T2B, 1), lambda i: (i, 0)),
        pl.BlockSpec((1, HID_DIM), lambda i: (0, 0)),
        pl.BlockSpec((HID_DIM, 1), lambda i: (0, 0)),
    ],
    out_specs=pl.BlockSpec((_T2B,), lambda i: (i,)),
    out_shape=jax.ShapeDtypeStruct((N_PAD,), jnp.float32),
)


def _t3_body(agg2p_ref, y2_ref, dinv_ref, b2_ref, out_ref):
    a = (agg2p_ref[0, :N_NODES] + agg2p_ref[1, :N_NODES]
         + y2_ref[:N_NODES])
    out_ref[...] = dinv_ref[:, 0] * a + b2_ref[0, 0]


_t3 = pl.pallas_call(
    _t3_body,
    out_shape=jax.ShapeDtypeStruct((N_NODES,), jnp.float32),
)


def kernel(x, edge_index, W1, b1, W2, b2):
    ei4 = edge_index.reshape(2, NW, NCH, CHUNK)

    zeros32 = jnp.zeros((STRIPE, HID_DIM), jnp.float32)
    zeros1 = jnp.zeros((STRIPE,), jnp.float32)
    ones_c = jnp.ones((CHUNK,), jnp.float32)

    degp = _deg_kernel(ei4, ones_c, zeros1)
    y1, dinv = _t1(x, W1, degp)
    aggp = _agg32_kernel(y1, ei4, zeros32)
    y2 = _t2(aggp, y1, dinv, b1.reshape(1, HID_DIM), W2)
    agg2p = _agg1_kernel(y2, ei4, zeros1)
    out = _t3(agg2p, y2, dinv, b2.reshape(1, 1))
    return out


# speedup vs baseline: 104.6204x; 1.2011x over previous
import functools

import jax
import jax.numpy as jnp
from jax import lax
from jax.experimental import pallas as pl
from jax.experimental.pallas import tpu as pltpu
from jax.experimental.pallas import tpu_sc as plsc

N_NODES = 10000
N_EDGES = 320000
IN_DIM = 128
HID_DIM = 32

NC = 2
NS = 16
NW = NC * NS
N_PAD = 10240
STRIPE = N_PAD // NS
E_PER_W = N_EDGES // NW

CHUNK = 80
NCH = E_PER_W // CHUNK
WAVE = 5
NWAVES = NCH // WAVE
NGRP = 3
NBUF = NGRP * WAVE


def _sc_mesh():
    return plsc.VectorSubcoreMesh(core_axis_name="c", subcore_axis_name="s")


_SC_PARAMS = pltpu.CompilerParams(use_tc_tiling_on_sc=False,
                                  needs_layout_passes=False)


@functools.partial(
    pl.kernel,
    mesh=_sc_mesh(),
    out_type=jax.ShapeDtypeStruct((NC, N_PAD), jnp.float32),
    scratch_types=[
        pltpu.VMEM((NCH, CHUNK), jnp.int32),
        pltpu.VMEM((CHUNK,), jnp.float32),
        pltpu.VMEM_SHARED((N_PAD,), jnp.float32),
        pltpu.SemaphoreType.DMA,
    ],
    compiler_params=_SC_PARAMS,
)
def _deg_kernel(ei_hbm, ones_hbm, zeros_hbm, out_hbm, idx_d, ones_v, acc, sem):
    c = lax.axis_index("c")
    s = lax.axis_index("s")
    wid = c * NS + s
    row0 = pl.multiple_of(s * STRIPE, 8)
    pltpu.sync_copy(zeros_hbm, acc.at[pl.ds(row0, STRIPE)])
    pltpu.sync_copy(ones_hbm, ones_v)
    pltpu.sync_copy(ei_hbm.at[1, wid], idx_d)
    plsc.subcore_barrier()

    def fire(i, carry):
        pltpu.async_copy(ones_v, acc.at[idx_d.at[i]], sem, add=True)
        return carry

    def drain(i, carry):
        pltpu.make_async_copy(ones_v, acc.at[idx_d.at[i]], sem).wait()
        return carry

    lax.fori_loop(0, NCH, fire, 0)
    lax.fori_loop(0, NCH, drain, 0)
    plsc.subcore_barrier()
    pltpu.sync_copy(acc.at[pl.ds(row0, STRIPE)],
                    out_hbm.at[c, pl.ds(row0, STRIPE)])


@functools.partial(
    pl.kernel,
    mesh=_sc_mesh(),
    out_type=jax.ShapeDtypeStruct((NC, N_PAD, HID_DIM), jnp.float32),
    scratch_types=[
        pltpu.VMEM((NCH, CHUNK), jnp.int32),
        pltpu.VMEM((NCH, CHUNK), jnp.int32),
        pltpu.VMEM((NBUF, CHUNK, HID_DIM), jnp.float32),
        pltpu.VMEM_SHARED((N_PAD, HID_DIM), jnp.float32),
        pltpu.SemaphoreType.DMA,
        pltpu.SemaphoreType.DMA,
    ],
    compiler_params=_SC_PARAMS,
)
def _agg32_kernel(tab_hbm, ei_hbm, zeros_hbm, out_hbm,
                  idx_s, idx_d, rows, acc, sem_g, sem_s):
    c = lax.axis_index("c")
    s = lax.axis_index("s")
    wid = c * NS + s
    row0 = pl.multiple_of(s * STRIPE, 8)
    last_full = (N_NODES - 15 * STRIPE)

    @pl.when((c == 0) & (s < NS - 1))
    def _():
        pltpu.sync_copy(tab_hbm.at[pl.ds(row0, STRIPE)],
                        acc.at[pl.ds(row0, STRIPE)])

    @pl.when((c == 0) & (s == NS - 1))
    def _():
        pltpu.sync_copy(tab_hbm.at[pl.ds(15 * STRIPE, last_full)],
                        acc.at[pl.ds(15 * STRIPE, last_full)])
        pltpu.sync_copy(zeros_hbm.at[pl.ds(0, STRIPE - last_full)],
                        acc.at[pl.ds(15 * STRIPE + last_full,
                                     STRIPE - last_full)])

    @pl.when(c == 1)
    def _():
        pltpu.sync_copy(zeros_hbm, acc.at[pl.ds(row0, STRIPE)])

    pltpu.sync_copy(ei_hbm.at[0, wid], idx_s)
    pltpu.sync_copy(ei_hbm.at[1, wid], idx_d)
    plsc.subcore_barrier()

    def fire_gather(chunk, slot):
        pltpu.async_copy(tab_hbm.at[idx_s.at[chunk]], rows.at[slot], sem_g)

    def drain_gather(chunk, slot):
        pltpu.make_async_copy(tab_hbm.at[idx_s.at[chunk]], rows.at[slot],
                              sem_g).wait()

    def fire_scatter(chunk, slot):
        pltpu.async_copy(rows.at[slot], acc.at[idx_d.at[chunk]], sem_s,
                         add=True)

    def drain_scatter(chunk, slot):
        pltpu.make_async_copy(rows.at[slot], acc.at[idx_d.at[chunk]],
                              sem_s).wait()

    for k in range(WAVE):
        fire_gather(k, k)

    def wave(w, carry):
        g = lax.rem(w, NGRP)
        ng = lax.rem(w + 1, NGRP)
        slot0 = g * WAVE
        nslot0 = ng * WAVE
        base = w * WAVE

        @pl.when(w > 1)
        def _():
            for k in range(WAVE):
                drain_scatter(base - 2 * WAVE + k, nslot0 + k)

        @pl.when(w < NWAVES - 1)
        def _():
            for k in range(WAVE):
                fire_gather(base + WAVE + k, nslot0 + k)

        for k in range(WAVE):
            drain_gather(base + k, slot0 + k)
        for k in range(WAVE):
            fire_scatter(base + k, slot0 + k)
        return carry

    lax.fori_loop(0, NWAVES, wave, 0)
    for w in (NWAVES - 2, NWAVES - 1):
        for k in range(WAVE):
            drain_scatter(w * WAVE + k, (w % NGRP) * WAVE + k)
    plsc.subcore_barrier()
    pltpu.sync_copy(acc.at[pl.ds(row0, STRIPE)],
                    out_hbm.at[c, pl.ds(row0, STRIPE)])


@functools.partial(
    pl.kernel,
    mesh=_sc_mesh(),
    out_type=jax.ShapeDtypeStruct((NC, N_PAD), jnp.float32),
    scratch_types=[
        pltpu.VMEM((NCH, CHUNK), jnp.int32),
        pltpu.VMEM((NCH, CHUNK), jnp.int32),
        pltpu.VMEM((N_PAD,), jnp.float32),
        pltpu.VMEM((E_PER_W,), jnp.float32),
        pltpu.VMEM_SHARED((N_PAD,), jnp.float32),
        pltpu.SemaphoreType.DMA,
    ],
    compiler_params=_SC_PARAMS,
)
def _agg1_kernel(tab_hbm, ei_hbm, zeros_hbm, out_hbm,
                 idx_s, idx_d, tab_v, vals, acc, sem):
    c = lax.axis_index("c")
    s = lax.axis_index("s")
    wid = c * NS + s
    row0 = pl.multiple_of(s * STRIPE, 8)
    pltpu.sync_copy(zeros_hbm, acc.at[pl.ds(row0, STRIPE)])
    pltpu.sync_copy(tab_hbm, tab_v)
    pltpu.sync_copy(ei_hbm.at[0, wid], idx_s)
    pltpu.sync_copy(ei_hbm.at[1, wid], idx_d)
    plsc.subcore_barrier()

    def gat(i, carry):
        for j in range(CHUNK // 16):
            iv = idx_s[i, pl.ds(j * 16, 16)]
            vals[pl.ds(i * CHUNK + j * 16, 16)] = plsc.load_gather(tab_v, [iv])
        return carry

    lax.fori_loop(0, NCH, gat, 0)

    def fire(i, carry):
        pltpu.async_copy(vals.at[pl.ds(i * CHUNK, CHUNK)],
                         acc.at[idx_d.at[i]], sem, add=True)
        return carry

    def drain(i, carry):
        pltpu.make_async_copy(vals.at[pl.ds(i * CHUNK, CHUNK)],
                              acc.at[idx_d.at[i]], sem).wait()
        return carry

    lax.fori_loop(0, NCH, fire, 0)
    lax.fori_loop(0, NCH, drain, 0)
    plsc.subcore_barrier()
    pltpu.sync_copy(acc.at[pl.ds(row0, STRIPE)],
                    out_hbm.at[c, pl.ds(row0, STRIPE)])


def _t1_body(x_ref, w1_ref, degp_ref, y1_ref):
    deg = degp_ref[0, :N_NODES] + degp_ref[1, :N_NODES] + 1.0
    dinv = lax.rsqrt(deg)
    h = jnp.dot(x_ref[...], w1_ref[...], preferred_element_type=jnp.float32)
    y1_ref[...] = h * dinv[:, None]


_t1 = pl.pallas_call(
    _t1_body,
    out_shape=jax.ShapeDtypeStruct((N_NODES, HID_DIM), jnp.float32),
)


_PROW = N_PAD // 4


def _t2_body(aggp_ref, degp_ref, b1p_ref, w2b_ref, y2_ref):
    tot = aggp_ref[:_PROW, :] + aggp_ref[_PROW:, :]
    deg4 = degp_ref[0] + degp_ref[1] + 1.0
    dinv4 = lax.rsqrt(deg4)
    scale = jnp.concatenate(
        [jnp.broadcast_to(dinv4[:, k:k + 1], (_PROW, HID_DIM))
         for k in range(4)], axis=1)
    x2 = jnp.maximum(scale * tot + b1p_ref[...], 0.0)
    g4 = jnp.dot(x2, w2b_ref[...], preferred_element_type=jnp.float32)
    y2_ref[...] = dinv4 * g4


_t2 = pl.pallas_call(
    _t2_body,
    out_shape=jax.ShapeDtypeStruct((_PROW, 4), jnp.float32),
)


_PR128 = N_PAD // 128


def _t3_body(agg2p_ref, y2p_ref, degp_ref, b2_ref, out_ref):
    deg = degp_ref[:_PR128, :] + degp_ref[_PR128:, :] + 1.0
    dinv = lax.rsqrt(deg)
    a = agg2p_ref[:_PR128, :] + agg2p_ref[_PR128:, :] + y2p_ref[...]
    out_ref[...] = dinv * a + b2_ref[0, 0]


_t3 = pl.pallas_call(
    _t3_body,
    out_shape=jax.ShapeDtypeStruct((_PR128, 128), jnp.float32),
)


def kernel(x, edge_index, W1, b1, W2, b2):
    ei4 = edge_index.reshape(2, NW, NCH, CHUNK)

    zeros32 = jnp.zeros((STRIPE, HID_DIM), jnp.float32)
    zeros1 = jnp.zeros((STRIPE,), jnp.float32)
    ones_c = jnp.ones((CHUNK,), jnp.float32)

    b1p = jnp.tile(b1, 4).reshape(1, 4 * HID_DIM)
    w2b = jnp.zeros((4 * HID_DIM, 4), jnp.float32)
    for k in range(4):
        w2b = w2b.at[k * HID_DIM:(k + 1) * HID_DIM, k].set(W2[:, 0])

    degp = _deg_kernel(ei4, ones_c, zeros1)
    y1 = _t1(x, W1, degp)
    aggp = _agg32_kernel(y1, ei4, zeros32)
    y2_4 = _t2(aggp.reshape(2 * _PROW, 4 * HID_DIM),
               degp.reshape(2, _PROW, 4), b1p, w2b)
    y2 = y2_4.reshape(-1)
    agg2p = _agg1_kernel(y2, ei4, zeros1)
    outp = _t3(agg2p.reshape(2 * _PR128, 128), y2.reshape(_PR128, 128),
               degp.reshape(2 * _PR128, 128), b2.reshape(1, 1))
    return outp.reshape(-1)[:N_NODES]
